# baseline (speedup 1.0000x reference)
import functools

import jax
import jax.numpy as jnp
from jax import lax
from jax.experimental import pallas as pl
from jax.experimental.pallas import tpu as pltpu
from jax.experimental.pallas import tpu_sc as plsc

DIM = 64

_info = plsc.get_sparse_core_info()
_NC, _NS = _info.num_cores, _info.num_subcores
_NW = _NC * _NS
_L = 16


@functools.cache
def _make(R: int, S: int, D: int):
    IB = 128
    KPW = R // (_NW * IB)
    n_ch = S * KPW
    assert (n_ch - D) % D == 0
    mesh = plsc.VectorSubcoreMesh(core_axis_name="c", subcore_axis_name="s")

    @functools.partial(
        pl.kernel,
        out_type=jax.ShapeDtypeStruct((S, 8, R // IB, 8, IB), jnp.float32),
        mesh=mesh,
        scratch_types=[
            pltpu.VMEM((S, KPW * IB), jnp.int32),
            [pltpu.VMEM((IB, DIM), jnp.float32) for _ in range(D)],
            [pltpu.VMEM((8, 8, IB), jnp.float32) for _ in range(D)],
            [pltpu.SemaphoreType.DMA for _ in range(D)],
            [pltpu.SemaphoreType.DMA for _ in range(D)],
        ],
        compiler_params=pltpu.CompilerParams(
            use_tc_tiling_on_sc=False, needs_layout_passes=False),
    )
    def gather_kernel(idx_hbm, table_hbm, out5, idx_v, gs, gts, sg, sw):
        wid = lax.axis_index("s") * _NC + lax.axis_index("c")
        k0 = wid * KPW
        pltpu.sync_copy(idx_hbm.at[:, pl.ds(k0 * IB, KPW * IB)], idx_v)

        rows_idx = [lax.iota(jnp.int32, _L) + _L * p for p in range(IB // _L)]

        def offsets(tc):
            j = tc // KPW
            kb = tc - j * KPW
            return idx_v.at[j, pl.ds(kb * IB, IB)]

        def gather(tc, b):
            return pltpu.async_copy(table_hbm.at[offsets(tc)], gs[b], sg[b])

        def gather_wait(tc, b):
            pltpu.make_async_copy(table_hbm.at[offsets(tc)], gs[b],
                                  sg[b]).wait()

        def dst(tc):
            j = tc // KPW
            kb = tc - j * KPW
            return out5.at[j, :, k0 + kb, :, :]

        def write(tc, b):
            return pltpu.async_copy(gts[b], dst(tc), sw[b])

        def write_wait(tc, b):
            pltpu.make_async_copy(gts[b], dst(tc), sw[b]).wait()

        def transpose(b):
            g, gt = gs[b], gts[b]
            for blk in range(8):
                for r in range(8):
                    d = jnp.full((_L,), 8 * blk + r, jnp.int32)
                    for p in range(IB // _L):
                        gt[blk, r, pl.ds(_L * p, _L)] = plsc.load_gather(
                            g, [rows_idx[p], d])

        for b in range(D):
            gather(b, b)

        @pl.loop(0, n_ch - D, step=D)
        def _(t):
            for b in range(D):
                tc = t + b

                @pl.when(tc >= D)
                def _():
                    write_wait(tc - D, b)

                gather_wait(tc, b)
                transpose(b)
                write(tc, b)
                gather(tc + D, b)

        for b in range(D):
            tc = n_ch - D + b
            write_wait(tc - D, b)
            gather_wait(tc, b)
            transpose(b)
            write(tc, b)
        for b in range(D):
            write_wait(n_ch - D + b, b)

    return gather_kernel


def kernel(inputs, embedding_table):
    R, S = inputs.shape
    idx_t = inputs.T.astype(jnp.int32)
    v5 = _make(R, S, 2)(idx_t, embedding_table)
    return v5.transpose(2, 4, 0, 1, 3).reshape(R, S, DIM)


# speedup vs baseline: 1.8109x; 1.8109x over previous
import functools

import jax
import jax.numpy as jnp
from jax import lax
from jax.experimental import pallas as pl
from jax.experimental.pallas import tpu as pltpu
from jax.experimental.pallas import tpu_sc as plsc

DIM = 64

_info = plsc.get_sparse_core_info()
_NC, _NS = _info.num_cores, _info.num_subcores
_NW = _NC * _NS
_L = 16


@functools.cache
def _make(R: int, S: int, D: int):
    IB = 128
    KPW = R // (_NW * IB)
    n_ch = S * KPW
    assert (n_ch - D) % D == 0
    mesh = plsc.VectorSubcoreMesh(core_axis_name="c", subcore_axis_name="s")

    @functools.partial(
        pl.kernel,
        out_type=jax.ShapeDtypeStruct((S, 8, R // IB, 8, IB), jnp.float32),
        mesh=mesh,
        scratch_types=[
            pltpu.VMEM((S, KPW * IB), jnp.int32),
            [pltpu.VMEM((IB, DIM), jnp.float32) for _ in range(D)],
            [pltpu.VMEM((8, 8, IB + 1), jnp.float32) for _ in range(D)],
            [pltpu.SemaphoreType.DMA for _ in range(D)],
            [pltpu.SemaphoreType.DMA for _ in range(D)],
        ],
        compiler_params=pltpu.CompilerParams(
            use_tc_tiling_on_sc=False, needs_layout_passes=False),
    )
    def gather_kernel(idx_hbm, table_hbm, out5, idx_v, gs, gts, sg, sw):
        wid = lax.axis_index("s") * _NC + lax.axis_index("c")
        k0 = wid * KPW
        pltpu.sync_copy(idx_hbm.at[:, pl.ds(k0 * IB, KPW * IB)], idx_v)

        iota = lax.iota(jnp.int32, _L)
        blk_q = [(_L * q + iota) // 8 for q in range(DIM // _L)]
        r_q = [(_L * q + iota) % 8 for q in range(DIM // _L)]

        def offsets(tc):
            j = tc // KPW
            kb = tc - j * KPW
            return idx_v.at[j, pl.ds(kb * IB, IB)]

        def gather(tc, b):
            return pltpu.async_copy(table_hbm.at[offsets(tc)], gs[b], sg[b])

        def gather_wait(tc, b):
            pltpu.make_async_copy(table_hbm.at[offsets(tc)], gs[b],
                                  sg[b]).wait()

        def dst(tc):
            j = tc // KPW
            kb = tc - j * KPW
            return out5.at[j, :, k0 + kb, :, :]

        def write(tc, b):
            return pltpu.async_copy(gts[b].at[:, :, pl.ds(0, IB)], dst(tc),
                                    sw[b])

        def write_wait(tc, b):
            pltpu.make_async_copy(gts[b].at[:, :, pl.ds(0, IB)], dst(tc),
                                  sw[b]).wait()

        def transpose(b):
            g, gt = gs[b], gts[b]
            for p in range(IB):
                c = jnp.full((_L,), p, jnp.int32)
                for q in range(DIM // _L):
                    plsc.store_scatter(gt, [blk_q[q], r_q[q], c],
                                       g[p, pl.ds(_L * q, _L)])

        for b in range(D):
            gather(b, b)

        @pl.loop(0, n_ch - D, step=D)
        def _(t):
            for b in range(D):
                tc = t + b

                @pl.when(tc >= D)
                def _():
                    write_wait(tc - D, b)

                gather_wait(tc, b)
                transpose(b)
                write(tc, b)
                gather(tc + D, b)

        for b in range(D):
            tc = n_ch - D + b
            write_wait(tc - D, b)
            gather_wait(tc, b)
            transpose(b)
            write(tc, b)
        for b in range(D):
            write_wait(n_ch - D + b, b)

    return gather_kernel


def kernel(inputs, embedding_table):
    R, S = inputs.shape
    idx_t = inputs.T.astype(jnp.int32)
    v5 = _make(R, S, 2)(idx_t, embedding_table)
    return v5.transpose(2, 4, 0, 1, 3).reshape(R, S, DIM)
